# Initial kernel scaffold; baseline (speedup 1.0000x reference)
import jax
import jax.numpy as jnp
from jax.experimental import pallas as pl


def kernel(x, embedding_weight):
    raise NotImplementedError("write your pallas kernel here")


# speedup vs baseline: 1.5200x; 1.5200x over previous
import functools

import jax
import jax.numpy as jnp
from jax.experimental import pallas as pl


CODEBOOK_SIZE = 1024
CODE_DIM = 64
COMMITMENT_WEIGHT = 0.25


def _vq_kernel(x_ref, emb_ref, q_ref, idx_ref, loss_ref, *, n_steps):
    xt = x_ref[0]
    emb = emb_ref[...]
    flat = xt.T

    scores = jax.lax.dot_general(
        flat.astype(jnp.bfloat16), emb, (((1,), (1,)), ((), ())),
        preferred_element_type=jnp.float32)
    x_sq = jnp.sum(flat * flat, axis=1, keepdims=True)
    e_sq = jnp.sum(emb * emb, axis=1)
    dist = x_sq - 2.0 * scores + e_sq[None, :]

    col = jax.lax.broadcasted_iota(jnp.int32, dist.shape, 1)
    m = jnp.min(dist, axis=1, keepdims=True)
    cand = jnp.where(dist == m, col, jnp.int32(CODEBOOK_SIZE))
    idx = jnp.min(cand, axis=1)
    idx_ref[0, 0] = idx

    onehot = (col == idx[:, None]).astype(jnp.float32)
    q = jax.lax.dot_general(
        onehot, emb, (((1,), (0,)), ((), ())),
        preferred_element_type=jnp.float32).T

    diff = q - xt
    q_ref[0] = xt + diff
    partial = jnp.sum(diff * diff, keepdims=True)

    step = pl.program_id(0) * pl.num_programs(1) + pl.program_id(1)

    @pl.when(step == 0)
    def _init():
        loss_ref[...] = jnp.zeros_like(loss_ref)

    loss_ref[...] += partial


def kernel(x, embedding_weight):
    bsz, channels, steps = x.shape
    T = 512
    n_t = steps // T
    grid = (bsz, n_t)

    q, idx3, loss = pl.pallas_call(
        functools.partial(_vq_kernel, n_steps=bsz * n_t),
        grid=grid,
        in_specs=[
            pl.BlockSpec((1, channels, T), lambda b, t: (b, 0, t)),
            pl.BlockSpec((CODEBOOK_SIZE, CODE_DIM), lambda b, t: (0, 0)),
        ],
        out_specs=[
            pl.BlockSpec((1, channels, T), lambda b, t: (b, 0, t)),
            pl.BlockSpec((1, 1, T), lambda b, t: (b, 0, t)),
            pl.BlockSpec((1, 1), lambda b, t: (0, 0)),
        ],
        out_shape=[
            jax.ShapeDtypeStruct((bsz, channels, steps), jnp.float32),
            jax.ShapeDtypeStruct((bsz, 1, steps), jnp.int32),
            jax.ShapeDtypeStruct((1, 1), jnp.float32),
        ],
    )(x, embedding_weight)

    scale = (1.0 + COMMITMENT_WEIGHT) / x.size
    return q, idx3.reshape(bsz, steps), (loss[0, 0] * scale).astype(jnp.float32)


# speedup vs baseline: 2.1744x; 1.4305x over previous
import functools

import jax
import jax.numpy as jnp
from jax.experimental import pallas as pl
from jax.experimental.pallas import tpu as pltpu


CODEBOOK_SIZE = 1024
CODE_DIM = 64
COMMITMENT_WEIGHT = 0.25


def _vq_kernel(x_ref, emb_ref, q_ref, idx_ref, loss_ref, esq_ref):
    t_step = pl.program_id(1)
    xt = x_ref[0]
    emb = emb_ref[...]

    @pl.when(t_step == 0)
    def _per_batch_init():
        esq_ref[...] = jnp.sum(emb * emb, axis=1, keepdims=True)
        loss_ref[...] = jnp.zeros_like(loss_ref)

    x16 = xt.astype(jnp.bfloat16)
    scores = jax.lax.dot_general(
        emb, x16, (((1,), (0,)), ((), ())),
        preferred_element_type=jnp.float32)
    x_sq = jnp.sum(xt * xt, axis=0, keepdims=True)
    dist = x_sq - 2.0 * scores + esq_ref[...]

    row = jax.lax.broadcasted_iota(jnp.int32, dist.shape, 0)
    m = jnp.min(dist, axis=0, keepdims=True)
    cand = jnp.where(dist == m, row, jnp.int32(CODEBOOK_SIZE))
    idx = jnp.min(cand, axis=0)
    idx_ref[0, 0] = idx

    onehot = (row == idx[None, :]).astype(jnp.float32)
    q = jax.lax.dot_general(
        emb, onehot, (((0,), (0,)), ((), ())),
        preferred_element_type=jnp.float32)

    diff = q - xt
    q_ref[0] = xt + diff
    loss_ref[...] += jnp.sum(diff * diff, keepdims=True).reshape(1, 1, 1)


def kernel(x, embedding_weight):
    bsz, channels, steps = x.shape
    T = 512
    grid = (bsz, steps // T)

    q, idx3, loss = pl.pallas_call(
        _vq_kernel,
        grid=grid,
        in_specs=[
            pl.BlockSpec((1, channels, T), lambda b, t: (b, 0, t)),
            pl.BlockSpec((CODEBOOK_SIZE, CODE_DIM), lambda b, t: (0, 0)),
        ],
        out_specs=[
            pl.BlockSpec((1, channels, T), lambda b, t: (b, 0, t)),
            pl.BlockSpec((1, 1, T), lambda b, t: (b, 0, t)),
            pl.BlockSpec((1, 1, 1), lambda b, t: (b, 0, 0)),
        ],
        out_shape=[
            jax.ShapeDtypeStruct((bsz, channels, steps), jnp.float32),
            jax.ShapeDtypeStruct((bsz, 1, steps), jnp.int32),
            jax.ShapeDtypeStruct((bsz, 1, 1), jnp.float32),
        ],
        scratch_shapes=[pltpu.VMEM((CODEBOOK_SIZE, 1), jnp.float32)],
        compiler_params=pltpu.CompilerParams(
            dimension_semantics=("parallel", "arbitrary")),
    )(x, embedding_weight)

    scale = (1.0 + COMMITMENT_WEIGHT) / x.size
    return q, idx3.reshape(bsz, steps), (jnp.sum(loss) * scale).astype(jnp.float32)


# speedup vs baseline: 2.7569x; 1.2679x over previous
import functools

import jax
import jax.numpy as jnp
from jax.experimental import pallas as pl
from jax.experimental.pallas import tpu as pltpu


CODEBOOK_SIZE = 1024
CODE_DIM = 64
COMMITMENT_WEIGHT = 0.25


def _vq_kernel(x_ref, emb_ref, q_ref, idx_ref, loss_ref, esq_ref):
    t_step = pl.program_id(1)
    xt = x_ref[0]
    emb = emb_ref[...]

    @pl.when(t_step == 0)
    def _per_batch_init():
        esq_ref[...] = jnp.sum(emb * emb, axis=1, keepdims=True)
        loss_ref[...] = jnp.zeros_like(loss_ref)

    x16 = xt.astype(jnp.bfloat16)
    scores2 = jax.lax.dot_general(
        emb + emb, x16, (((1,), (0,)), ((), ())),
        preferred_element_type=jnp.float32)
    x_sq = jnp.sum(xt * xt, axis=0, keepdims=True)
    dist = x_sq - scores2 + esq_ref[...]

    row = jax.lax.broadcasted_iota(jnp.int32, dist.shape, 0)
    m = jnp.min(dist, axis=0, keepdims=True)
    cand = jnp.where(dist == m, row, jnp.int32(CODEBOOK_SIZE))
    idx = jnp.min(cand, axis=0)
    idx_ref[0, 0] = idx

    onehot = (row == idx[None, :]).astype(jnp.float32)
    q = jax.lax.dot_general(
        emb, onehot, (((0,), (0,)), ((), ())),
        preferred_element_type=jnp.float32)

    diff = q - xt
    q_ref[0] = xt + diff
    loss_ref[...] += jnp.sum(diff * diff, keepdims=True).reshape(1, 1, 1)


def kernel(x, embedding_weight):
    bsz, channels, steps = x.shape
    T = 1024
    grid = (bsz, steps // T)

    q, idx3, loss = pl.pallas_call(
        _vq_kernel,
        grid=grid,
        in_specs=[
            pl.BlockSpec((1, channels, T), lambda b, t: (b, 0, t)),
            pl.BlockSpec((CODEBOOK_SIZE, CODE_DIM), lambda b, t: (0, 0)),
        ],
        out_specs=[
            pl.BlockSpec((1, channels, T), lambda b, t: (b, 0, t)),
            pl.BlockSpec((1, 1, T), lambda b, t: (b, 0, t)),
            pl.BlockSpec((1, 1, 1), lambda b, t: (b, 0, 0)),
        ],
        out_shape=[
            jax.ShapeDtypeStruct((bsz, channels, steps), jnp.float32),
            jax.ShapeDtypeStruct((bsz, 1, steps), jnp.int32),
            jax.ShapeDtypeStruct((bsz, 1, 1), jnp.float32),
        ],
        scratch_shapes=[pltpu.VMEM((CODEBOOK_SIZE, 1), jnp.float32)],
        compiler_params=pltpu.CompilerParams(
            dimension_semantics=("parallel", "arbitrary")),
    )(x, embedding_weight)

    scale = (1.0 + COMMITMENT_WEIGHT) / x.size
    return q, idx3.reshape(bsz, steps), (jnp.sum(loss) * scale).astype(jnp.float32)
